# Initial kernel scaffold; baseline (speedup 1.0000x reference)
import jax
import jax.numpy as jnp
from jax.experimental import pallas as pl


def kernel(x, edge_index, edge_attr, ptr, emb, W1l, b1l, W1r, b1r, W1e, att1, bias1, W2l, b2l, W2r, b2r, W2e, att2, bias2, W3l, b3l, W3r, b3r, W3e, att3, bias3):
    raise NotImplementedError("write your pallas kernel here")


# speedup vs baseline: 2.7553x; 2.7553x over previous
import jax
import jax.numpy as jnp
from jax.experimental import pallas as pl
from jax.experimental.pallas import tpu as pltpu

N = 50000
NB = 16
_BN = 512
_NP = ((N + _BN - 1) // _BN) * _BN


def _pool_body(lo_ref, hi_ref, h_ref, out_ref, sums, cnts):
    i = pl.program_id(0)

    @pl.when(i == 0)
    def _():
        sums[...] = jnp.zeros_like(sums)
        cnts[...] = jnp.zeros_like(cnts)

    idx = jax.lax.broadcasted_iota(jnp.int32, (NB, _BN), 1) + i * _BN
    lo = jnp.broadcast_to(lo_ref[:, 0:1], (NB, _BN))
    hi = jnp.broadcast_to(hi_ref[:, 0:1], (NB, _BN))
    m = jnp.logical_and(idx >= lo, idx < hi).astype(jnp.float32)
    sums[...] += jnp.dot(m, h_ref[...], preferred_element_type=jnp.float32)
    cnts[...] += jnp.broadcast_to(jnp.sum(m, axis=1, keepdims=True), cnts.shape)

    @pl.when(i == pl.num_programs(0) - 1)
    def _():
        c = jnp.maximum(cnts[:, 0:1], 1.0)
        out_ref[...] = sums[...] / jnp.broadcast_to(c, out_ref.shape)


def _pool(h, ptr):
    hp = jnp.pad(h, ((0, _NP - N), (0, 0)))
    lo = jnp.broadcast_to(ptr[:NB, None], (NB, 128)).astype(jnp.int32)
    hi = jnp.broadcast_to(ptr[1:, None], (NB, 128)).astype(jnp.int32)
    return pl.pallas_call(
        _pool_body,
        grid=(_NP // _BN,),
        in_specs=[
            pl.BlockSpec((NB, 128), lambda i: (0, 0)),
            pl.BlockSpec((NB, 128), lambda i: (0, 0)),
            pl.BlockSpec((_BN, h.shape[1]), lambda i: (i, 0)),
        ],
        out_specs=pl.BlockSpec((NB, h.shape[1]), lambda i: (0, 0)),
        out_shape=jax.ShapeDtypeStruct((NB, h.shape[1]), jnp.float32),
        scratch_shapes=[
            pltpu.VMEM((NB, h.shape[1]), jnp.float32),
            pltpu.VMEM((NB, 128), jnp.float32),
        ],
    )(lo, hi, hp)


def _leaky(v):
    return jnp.where(v >= 0, v, 0.2 * v)


def _layer(xl, xr, e_edge, e_loop, att, bias, src, dst):
    m = _leaky(xl[src] + xr[dst] + e_edge)
    ex = jnp.exp((m * att).sum(-1))
    den = jax.ops.segment_sum(ex, dst, num_segments=N)
    num = jax.ops.segment_sum(ex[:, None] * xl[src], dst, num_segments=N)
    us = _leaky(xl + xr + e_loop)
    exs = jnp.exp((us * att).sum(-1))
    return (num + exs[:, None] * xl) / (den + exs)[:, None] + bias


def kernel(x, edge_index, edge_attr, ptr, emb,
           W1l, b1l, W1r, b1r, W1e, att1, bias1,
           W2l, b2l, W2r, b2r, W2e, att2, bias2,
           W3l, b3l, W3r, b3r, W3e, att3, bias3):
    src, dst = edge_index[0], edge_index[1]

    XL1 = emb @ W1l.T + b1l
    XR1 = emb @ W1r.T + b1r
    EW1 = emb @ W1e.T
    EW2 = emb @ W2e.T
    EW3 = emb @ W3e.T

    flat = dst * 128 + edge_attr
    C = jnp.zeros((N * 128,), jnp.float32).at[flat].add(1.0).reshape(N, 128)
    cnt = jnp.maximum(C.sum(1), 1.0)[:, None]

    xl = XL1[x]
    xr = XR1[x]
    h = _layer(xl, xr, EW1[edge_attr], (C @ EW1) / cnt, att1, bias1, src, dst)
    h = jax.nn.elu(h)

    xl = h @ W2l.T + b2l
    xr = h @ W2r.T + b2r
    h = _layer(xl, xr, EW2[edge_attr], (C @ EW2) / cnt, att2, bias2, src, dst)
    h = jax.nn.elu(h)

    xl = h @ W3l.T + b3l
    xr = h @ W3r.T + b3r
    h = _layer(xl, xr, EW3[edge_attr], (C @ EW3) / cnt, att3, bias3, src, dst)

    return _pool(h, ptr)


# speedup vs baseline: 2.7999x; 1.0162x over previous
import functools

import jax
import jax.numpy as jnp
from jax import lax
from jax.experimental import pallas as pl
from jax.experimental.pallas import tpu as pltpu
from jax.experimental.pallas import tpu_sc as plsc

N = 50000
E = 800000
NB = 16
ROWS = 64
SUBS = 800
NPAD = SUBS * ROWS
CH = 120
EPAD = E + 128

_BN = 512
_NP = ((N + _BN - 1) // _BN) * _BN


def _pool_body(lo_ref, hi_ref, h_ref, out_ref, sums, cnts):
    i = pl.program_id(0)

    @pl.when(i == 0)
    def _():
        sums[...] = jnp.zeros_like(sums)
        cnts[...] = jnp.zeros_like(cnts)

    idx = jax.lax.broadcasted_iota(jnp.int32, (NB, _BN), 1) + i * _BN
    lo = jnp.broadcast_to(lo_ref[:, 0:1], (NB, _BN))
    hi = jnp.broadcast_to(hi_ref[:, 0:1], (NB, _BN))
    m = jnp.logical_and(idx >= lo, idx < hi).astype(jnp.float32)
    sums[...] += jnp.dot(m, h_ref[...], preferred_element_type=jnp.float32)
    cnts[...] += jnp.broadcast_to(jnp.sum(m, axis=1, keepdims=True), cnts.shape)

    @pl.when(i == pl.num_programs(0) - 1)
    def _():
        c = jnp.maximum(cnts[:, 0:1], 1.0)
        out_ref[...] = sums[...] / jnp.broadcast_to(c, out_ref.shape)


def _pool(h, ptr):
    hp = jnp.pad(h, ((0, _NP - N), (0, 0)))
    lo = jnp.broadcast_to(ptr[:NB, None], (NB, 128)).astype(jnp.int32)
    hi = jnp.broadcast_to(ptr[1:, None], (NB, 128)).astype(jnp.int32)
    return pl.pallas_call(
        _pool_body,
        grid=(_NP // _BN,),
        in_specs=[
            pl.BlockSpec((NB, 128), lambda i: (0, 0)),
            pl.BlockSpec((NB, 128), lambda i: (0, 0)),
            pl.BlockSpec((_BN, h.shape[1]), lambda i: (i, 0)),
        ],
        out_specs=pl.BlockSpec((NB, h.shape[1]), lambda i: (0, 0)),
        out_shape=jax.ShapeDtypeStruct((NB, h.shape[1]), jnp.float32),
        scratch_shapes=[
            pltpu.VMEM((NB, h.shape[1]), jnp.float32),
            pltpu.VMEM((NB, 128), jnp.float32),
        ],
    )(lo, hi, hp)


def _count_sc(dsts, attrs, bnd):
    mesh = plsc.VectorSubcoreMesh(core_axis_name="c", subcore_axis_name="s")

    @functools.partial(
        pl.kernel,
        out_type=(jax.ShapeDtypeStruct((NPAD, 128), jnp.float32),),
        mesh=mesh,
        scratch_types=[
            pltpu.VMEM((144,), jnp.int32),
            pltpu.VMEM((144,), jnp.int32),
            pltpu.VMEM((824,), jnp.int32),
            pltpu.VMEM((ROWS, 128), jnp.float32),
        ],
    )
    def k(dsts_h, attrs_h, bnd_h, c_h, dbuf, abuf, bndb, cl):
        wid = lax.axis_index("s") * 2 + lax.axis_index("c")
        pltpu.sync_copy(bnd_h, bndb.at[pl.ds(0, 808)])
        zf = jnp.zeros((16,), jnp.float32)
        lanes_f = lax.iota(jnp.int32, 16).astype(jnp.float32)
        one = jnp.full((16,), 1.0, jnp.float32)

        def sub_body(j, _):
            si = wid * 25 + j
            base = pl.multiple_of(si * ROWS, ROWS)

            def zrow(r, _):
                for kk in range(8):
                    cl[r, pl.ds(kk * 16, 16)] = zf
                return _

            lax.fori_loop(0, ROWS, zrow, None)

            bv = bndb[pl.ds(si, 16)]
            b_lo = bv[0]
            b_hi = bv[1]
            nch = (b_hi - b_lo + CH - 1) // CH

            def chunk_body(c, _):
                start = b_lo + c * CH
                end = jnp.minimum(start + CH, b_hi)
                a0 = pl.multiple_of(jnp.bitwise_and(start, -8), 8)
                pltpu.sync_copy(dsts_h.at[pl.ds(a0, 128)],
                                dbuf.at[pl.ds(0, 128)])
                pltpu.sync_copy(attrs_h.at[pl.ds(a0, 128)],
                                abuf.at[pl.ds(0, 128)])

                def edge(e, _):
                    o = e - a0
                    d = dbuf[pl.ds(o, 16)][0]
                    a = abuf[pl.ds(o, 16)][0]
                    r = d - base
                    t = lanes_f - jnp.broadcast_to(
                        jnp.bitwise_and(a, 15).astype(jnp.float32), (16,))
                    ohc = jnp.maximum(zf, one - t * t)
                    plsc.addupdate(
                        cl.at[r, pl.ds(jnp.bitwise_and(a, -16), 16)], ohc)
                    return _

                lax.fori_loop(start, end, edge, None)
                return _

            lax.fori_loop(0, nch, chunk_body, None)
            pltpu.sync_copy(cl, c_h.at[pl.ds(base, ROWS)])
            return _

        lax.fori_loop(0, 25, sub_body, None)

    return k(dsts, attrs, bnd)[0]


def _leaky(v):
    return jnp.where(v >= 0, v, 0.2 * v)


def kernel(x, edge_index, edge_attr, ptr, emb,
           W1l, b1l, W1r, b1r, W1e, att1, bias1,
           W2l, b2l, W2r, b2r, W2e, att2, bias2,
           W3l, b3l, W3r, b3r, W3e, att3, bias3):
    src, dst = edge_index[0], edge_index[1]

    XL1 = emb @ W1l.T + b1l
    XR1 = emb @ W1r.T + b1r
    EW1 = emb @ W1e.T
    EW2 = emb @ W2e.T
    EW3 = emb @ W3e.T

    dst_s, perm = lax.sort_key_val(dst, jnp.arange(E, dtype=jnp.int32))
    attrs = jnp.pad(edge_attr[perm], (0, EPAD - E))
    dsts = jnp.pad(dst_s, (0, EPAD - E))
    probes = jnp.arange(0, NPAD + 1, ROWS, dtype=jnp.int32)
    bnd = jnp.pad(jnp.searchsorted(dst_s, probes).astype(jnp.int32), (0, 7))
    C = _count_sc(dsts, attrs, bnd)[:N]
    cnt = jnp.maximum(C.sum(1), 1.0)[:, None]

    def layer(xl, xr, EW, att, bias):
        e_edge = EW[edge_attr]
        m = _leaky(xl[src] + xr[dst] + e_edge)
        ex = jnp.exp((m * att).sum(-1))
        den = jax.ops.segment_sum(ex, dst, num_segments=N)
        num = jax.ops.segment_sum(ex[:, None] * xl[src], dst, num_segments=N)
        e_loop = (C @ EW) / cnt
        us = _leaky(xl + xr + e_loop)
        exs = jnp.exp((us * att).sum(-1))
        return (num + exs[:, None] * xl) / (den + exs)[:, None] + bias

    h = layer(XL1[x], XR1[x], EW1, att1, bias1)
    h = jax.nn.elu(h)
    h = layer(h @ W2l.T + b2l, h @ W2r.T + b2r, EW2, att2, bias2)
    h = jax.nn.elu(h)
    h = layer(h @ W3l.T + b3l, h @ W3r.T + b3r, EW3, att3, bias3)
    return _pool(h, ptr)
